# baseline (speedup 1.0000x reference)
import functools

import jax
import jax.numpy as jnp
from jax import lax
from jax.experimental import pallas as pl
from jax.experimental.pallas import tpu as pltpu
from jax.experimental.pallas import tpu_sc as plsc

BATCH = 4096
SEQ = 200
HIDDEN = 128

NW = 32
ROWS = BATCH * SEQ
ROWS_PER_W = ROWS // NW
SEQ_PER_W = ROWS_PER_W // SEQ
R_A, R_B = 104, 96
DEPTH = 4


def _build():
    mesh = plsc.VectorSubcoreMesh(core_axis_name="c", subcore_axis_name="s")

    @functools.partial(
        pl.kernel,
        out_type=jax.ShapeDtypeStruct((ROWS, HIDDEN), jnp.float32),
        mesh=mesh,
        scratch_types=[
            pltpu.VMEM((ROWS_PER_W,), jnp.int32),
            pltpu.VMEM_SHARED((SEQ, HIDDEN), jnp.float32),
        ] + [pltpu.VMEM((R_A, HIDDEN), jnp.float32) for _ in range(DEPTH)]
          + [pltpu.VMEM((R_B, HIDDEN), jnp.float32) for _ in range(DEPTH)]
          + [pltpu.SemaphoreType.DMA for _ in range(6 * DEPTH)],
    )
    def emb_kernel(ids_hbm, tok_hbm, pos_hbm, out_hbm, idx_v, pos_v, *bs):
        bufs = {"A": bs[:DEPTH], "B": bs[DEPTH:2 * DEPTH]}
        gsem = {"A": bs[2 * DEPTH:3 * DEPTH], "B": bs[3 * DEPTH:4 * DEPTH]}
        osem = {"A": bs[4 * DEPTH:5 * DEPTH], "B": bs[5 * DEPTH:6 * DEPTH]}
        psem = {"A": bs[6 * DEPTH:7 * DEPTH], "B": bs[7 * DEPTH:8 * DEPTH]}
        rows = {"A": R_A, "B": R_B}
        off = {"A": 0, "B": R_A}

        wid = lax.axis_index("s") * 2 + lax.axis_index("c")
        base = wid * ROWS_PER_W

        pltpu.sync_copy(ids_hbm.at[wid], idx_v)

        @pl.when(lax.axis_index("s") == 0)
        def _():
            pltpu.sync_copy(pos_hbm, pos_v)

        plsc.subcore_barrier()

        def pparts(kind, d):
            return (pos_v.at[pl.ds(off[kind], rows[kind])], bufs[kind][d],
                    psem[kind][d])

        def gparts(q, kind, d):
            src = tok_hbm.at[idx_v.at[pl.ds(q * SEQ + off[kind], rows[kind])]]
            return src, bufs[kind][d], gsem[kind][d]

        def fire_chain(q, kind, d):
            pltpu.async_copy(*pparts(kind, d))
            pltpu.make_async_copy(*pparts(kind, d)).wait()
            pltpu.async_copy(*gparts(q, kind, d), add=True)

        def process(q, kind, d):
            pltpu.make_async_copy(*gparts(q, kind, d)).wait()
            dst = out_hbm.at[pl.ds(base + q * SEQ + off[kind], rows[kind])]
            pltpu.async_copy(bufs[kind][d], dst, osem[kind][d])

            @pl.when(q + DEPTH < SEQ_PER_W)
            def _():
                pltpu.make_async_copy(bufs[kind][d], dst,
                                      osem[kind][d]).wait()
                fire_chain(q + DEPTH, kind, d)

        for d in range(DEPTH):
            for kind in ("A", "B"):
                fire_chain(d, kind, d)

        @pl.loop(0, SEQ_PER_W, step=DEPTH)
        def group(qb):
            for d in range(DEPTH):
                for kind in ("A", "B"):
                    process(qb + d, kind, d)

        for d in range(DEPTH):
            for kind in ("A", "B"):
                pltpu.make_async_copy(
                    bufs[kind][d],
                    out_hbm.at[pl.ds(base + off[kind], rows[kind])],
                    osem[kind][d]).wait()

    return emb_kernel


_emb = _build()


def kernel(input_ids, token_embedding, position_embedding):
    ids = input_ids.reshape(NW, ROWS_PER_W).astype(jnp.int32)
    out = _emb(ids, token_embedding, position_embedding)
    return out.reshape(BATCH, SEQ, HIDDEN)


# speedup vs baseline: 1.1044x; 1.1044x over previous
import functools

import jax
import jax.numpy as jnp
from jax import lax
from jax.experimental import pallas as pl
from jax.experimental.pallas import tpu as pltpu
from jax.experimental.pallas import tpu_sc as plsc

BATCH = 4096
SEQ = 200
HIDDEN = 128

NW = 32
ROWS = BATCH * SEQ
ROWS_PER_W = ROWS // NW
SEQ_PER_W = ROWS_PER_W // SEQ
G0 = 128
G1 = SEQ - G0
DEPTH = 4


def _build():
    mesh = plsc.VectorSubcoreMesh(core_axis_name="c", subcore_axis_name="s")

    @functools.partial(
        pl.kernel,
        out_type=jax.ShapeDtypeStruct((ROWS, HIDDEN), jnp.float32),
        mesh=mesh,
        scratch_types=[
            pltpu.VMEM((ROWS_PER_W,), jnp.int32),
            pltpu.VMEM_SHARED((SEQ, HIDDEN), jnp.float32),
        ] + [pltpu.VMEM((SEQ, HIDDEN), jnp.float32) for _ in range(DEPTH)]
          + [pltpu.SemaphoreType.DMA for _ in range(3 * DEPTH)],
    )
    def emb_kernel(ids_hbm, tok_hbm, pos_hbm, out_hbm, idx_v, pos_v, *bs):
        bufs = bs[:DEPTH]
        gsem = bs[DEPTH:2 * DEPTH]
        osem = bs[2 * DEPTH:3 * DEPTH]
        psem = bs[3 * DEPTH:4 * DEPTH]

        wid = lax.axis_index("s") * 2 + lax.axis_index("c")
        base = wid * ROWS_PER_W

        pltpu.sync_copy(ids_hbm.at[wid], idx_v)

        @pl.when(lax.axis_index("s") == 0)
        def _():
            pltpu.sync_copy(pos_hbm, pos_v)

        plsc.subcore_barrier()

        def gparts(q, d):
            return (
                (tok_hbm.at[idx_v.at[pl.ds(q * SEQ, G0)]],
                 bufs[d].at[pl.ds(0, G0)], gsem[d]),
                (tok_hbm.at[idx_v.at[pl.ds(q * SEQ + G0, G1)]],
                 bufs[d].at[pl.ds(G0, G1)], gsem[d]),
            )

        def fire_chain(q, d):
            pltpu.async_copy(pos_v, bufs[d], psem[d])
            pltpu.make_async_copy(pos_v, bufs[d], psem[d]).wait()
            for part in gparts(q, d):
                pltpu.async_copy(*part, add=True)

        def process(q, d):
            for part in gparts(q, d):
                pltpu.make_async_copy(*part).wait()
            dst = out_hbm.at[pl.ds(base + q * SEQ, SEQ)]
            pltpu.async_copy(bufs[d], dst, osem[d])

            @pl.when(q + DEPTH < SEQ_PER_W)
            def _():
                pltpu.make_async_copy(bufs[d], dst, osem[d]).wait()
                fire_chain(q + DEPTH, d)

        for d in range(DEPTH):
            fire_chain(d, d)

        @pl.loop(0, SEQ_PER_W, step=DEPTH)
        def group(qb):
            for d in range(DEPTH):
                process(qb + d, d)

        for d in range(DEPTH):
            pltpu.make_async_copy(
                bufs[d], out_hbm.at[pl.ds(base, SEQ)], osem[d]).wait()

    return emb_kernel


_emb = _build()


def kernel(input_ids, token_embedding, position_embedding):
    ids = input_ids.reshape(NW, ROWS_PER_W).astype(jnp.int32)
    out = _emb(ids, token_embedding, position_embedding)
    return out.reshape(BATCH, SEQ, HIDDEN)


# speedup vs baseline: 1.1369x; 1.0294x over previous
import functools

import jax
import jax.numpy as jnp
from jax import lax
from jax.experimental import pallas as pl
from jax.experimental.pallas import tpu as pltpu
from jax.experimental.pallas import tpu_sc as plsc

BATCH = 4096
SEQ = 200
HIDDEN = 128

NW = 32
ROWS = BATCH * SEQ
ROWS_PER_W = ROWS // NW
PAIR = 2 * SEQ
PAIR_PER_W = ROWS_PER_W // PAIR
GSPLIT = (104, 104, 104, 88)
DEPTH = 2


def _build():
    mesh = plsc.VectorSubcoreMesh(core_axis_name="c", subcore_axis_name="s")

    @functools.partial(
        pl.kernel,
        out_type=jax.ShapeDtypeStruct((ROWS, HIDDEN), jnp.float32),
        mesh=mesh,
        scratch_types=[
            pltpu.VMEM((ROWS_PER_W,), jnp.int32),
            pltpu.VMEM_SHARED((SEQ, HIDDEN), jnp.float32),
        ] + [pltpu.VMEM((PAIR, HIDDEN), jnp.float32) for _ in range(DEPTH)]
          + [pltpu.SemaphoreType.DMA for _ in range(3 * DEPTH)],
    )
    def emb_kernel(ids_hbm, tok_hbm, pos_hbm, out_hbm, idx_v, pos_v, *bs):
        bufs = bs[:DEPTH]
        gsem = bs[DEPTH:2 * DEPTH]
        osem = bs[2 * DEPTH:3 * DEPTH]
        psem = bs[3 * DEPTH:4 * DEPTH]

        wid = lax.axis_index("s") * 2 + lax.axis_index("c")
        base = wid * ROWS_PER_W

        pltpu.sync_copy(ids_hbm.at[wid], idx_v)

        @pl.when(lax.axis_index("s") == 0)
        def _():
            pltpu.sync_copy(pos_hbm, pos_v)

        plsc.subcore_barrier()

        def pparts(d):
            return ((pos_v, bufs[d].at[pl.ds(0, SEQ)], psem[d]),
                    (pos_v, bufs[d].at[pl.ds(SEQ, SEQ)], psem[d]))

        def fire_prefill(d):
            for part in pparts(d):
                pltpu.async_copy(*part)

        def wait_prefill(d):
            for part in pparts(d):
                pltpu.make_async_copy(*part).wait()

        def gparts(p, d):
            parts = []
            o = 0
            for n in GSPLIT:
                parts.append((tok_hbm.at[idx_v.at[pl.ds(p * PAIR + o, n)]],
                              bufs[d].at[pl.ds(o, n)], gsem[d]))
                o += n
            return parts

        def fire_gathers(p, d):
            for part in gparts(p, d):
                pltpu.async_copy(*part, add=True)

        def process(p, d):
            for part in gparts(p, d):
                pltpu.make_async_copy(*part).wait()
            dst = out_hbm.at[pl.ds(base + p * PAIR, PAIR)]
            pltpu.async_copy(bufs[d], dst, osem[d])

            @pl.when(p + DEPTH < PAIR_PER_W)
            def _():
                pltpu.make_async_copy(bufs[d], dst, osem[d]).wait()
                fire_prefill(d)
                wait_prefill(d)
                fire_gathers(p + DEPTH, d)

        for d in range(DEPTH):
            fire_prefill(d)
        for d in range(DEPTH):
            wait_prefill(d)
            fire_gathers(d, d)

        @pl.loop(0, PAIR_PER_W, step=DEPTH)
        def group(pb):
            for d in range(DEPTH):
                process(pb + d, d)

        for d in range(DEPTH):
            pltpu.make_async_copy(
                bufs[d], out_hbm.at[pl.ds(base, PAIR)], osem[d]).wait()

    return emb_kernel


_emb = _build()


def kernel(input_ids, token_embedding, position_embedding):
    ids = input_ids.reshape(NW, ROWS_PER_W).astype(jnp.int32)
    out = _emb(ids, token_embedding, position_embedding)
    return out.reshape(BATCH, SEQ, HIDDEN)


# speedup vs baseline: 1.1383x; 1.0012x over previous
import functools

import jax
import jax.numpy as jnp
from jax import lax
from jax.experimental import pallas as pl
from jax.experimental.pallas import tpu as pltpu
from jax.experimental.pallas import tpu_sc as plsc

BATCH = 4096
SEQ = 200
HIDDEN = 128

NW = 32
ROWS = BATCH * SEQ
ROWS_PER_W = ROWS // NW
PAIR = 2 * SEQ
PAIR_PER_W = ROWS_PER_W // PAIR
DEPTH = 2


def _build():
    mesh = plsc.VectorSubcoreMesh(core_axis_name="c", subcore_axis_name="s")

    @functools.partial(
        pl.kernel,
        out_type=jax.ShapeDtypeStruct((ROWS, HIDDEN), jnp.float32),
        mesh=mesh,
        scratch_types=[
            pltpu.VMEM((ROWS_PER_W,), jnp.int32),
            pltpu.VMEM_SHARED((SEQ, HIDDEN), jnp.float32),
        ] + [pltpu.VMEM((PAIR, HIDDEN), jnp.float32) for _ in range(DEPTH)]
          + [pltpu.SemaphoreType.DMA for _ in range(3 * DEPTH)],
    )
    def emb_kernel(ids_hbm, tok_hbm, pos_hbm, out_hbm, idx_v, pos_v, *bs):
        bufs = bs[:DEPTH]
        gsem = bs[DEPTH:2 * DEPTH]
        osem = bs[2 * DEPTH:3 * DEPTH]
        psem = bs[3 * DEPTH:4 * DEPTH]

        wid = lax.axis_index("s") * 2 + lax.axis_index("c")
        base = wid * ROWS_PER_W

        pltpu.sync_copy(ids_hbm.at[wid], idx_v)

        @pl.when(lax.axis_index("s") == 0)
        def _():
            pltpu.sync_copy(pos_hbm, pos_v)

        plsc.subcore_barrier()

        def pparts(d):
            return ((pos_v, bufs[d].at[pl.ds(0, SEQ)], psem[d]),
                    (pos_v, bufs[d].at[pl.ds(SEQ, SEQ)], psem[d]))

        def fire_prefill(d):
            for part in pparts(d):
                pltpu.async_copy(*part)

        def wait_prefill(d):
            for part in pparts(d):
                pltpu.make_async_copy(*part).wait()

        def gparts(p, d):
            return (tok_hbm.at[idx_v.at[pl.ds(p * PAIR, PAIR)]], bufs[d],
                    gsem[d])

        def process(p, d):
            pltpu.make_async_copy(*gparts(p, d)).wait()
            dst = out_hbm.at[pl.ds(base + p * PAIR, PAIR)]
            pltpu.async_copy(bufs[d], dst, osem[d])

            @pl.when(p + DEPTH < PAIR_PER_W)
            def _():
                pltpu.make_async_copy(bufs[d], dst, osem[d]).wait()
                fire_prefill(d)
                wait_prefill(d)
                pltpu.async_copy(*gparts(p + DEPTH, d), add=True)

        for d in range(DEPTH):
            fire_prefill(d)
        for d in range(DEPTH):
            wait_prefill(d)
            pltpu.async_copy(*gparts(d, d), add=True)

        @pl.loop(0, PAIR_PER_W, step=DEPTH)
        def group(pb):
            for d in range(DEPTH):
                process(pb + d, d)

        for d in range(DEPTH):
            pltpu.make_async_copy(
                bufs[d], out_hbm.at[pl.ds(base, PAIR)], osem[d]).wait()

    return emb_kernel


_emb = _build()


def kernel(input_ids, token_embedding, position_embedding):
    ids = input_ids.reshape(NW, ROWS_PER_W).astype(jnp.int32)
    out = _emb(ids, token_embedding, position_embedding)
    return out.reshape(BATCH, SEQ, HIDDEN)
